# baseline (speedup 1.0000x reference)
import jax
import jax.numpy as jnp
from jax.experimental import pallas as pl

_SB = 4


def _posadd_kernel(x_ref, pos_ref, o_ref):
    i = pl.program_id(0)
    pos = pos_ref[pl.ds(i * _SB, _SB), :]
    o_ref[...] = x_ref[...] + pos[:, None, :]


def kernel(x, pos_table):
    B, S, D = x.shape
    xt = jnp.transpose(x, (1, 0, 2))
    out_t = pl.pallas_call(
        _posadd_kernel,
        grid=(pl.cdiv(S, _SB),),
        in_specs=[
            pl.BlockSpec((_SB, B, D), lambda i: (i, 0, 0)),
            pl.BlockSpec((512, D), lambda i: (0, 0)),
        ],
        out_specs=pl.BlockSpec((_SB, B, D), lambda i: (i, 0, 0)),
        out_shape=jax.ShapeDtypeStruct((S, B, D), x.dtype),
    )(xt, pos_table)
    return jnp.transpose(out_t, (1, 0, 2))


# speedup vs baseline: 1.1077x; 1.1077x over previous
import jax
import jax.numpy as jnp
from jax.experimental import pallas as pl

_SB = 16


def _posadd_kernel(x_ref, pos_ref, o_ref):
    i = pl.program_id(0)
    pos = pos_ref[pl.ds(i * _SB, _SB), :]
    o_ref[...] = x_ref[...] + pos[:, None, :]


def kernel(x, pos_table):
    B, S, D = x.shape
    xt = jnp.transpose(x, (1, 0, 2))
    out_t = pl.pallas_call(
        _posadd_kernel,
        grid=(pl.cdiv(S, _SB),),
        in_specs=[
            pl.BlockSpec((_SB, B, D), lambda i: (i, 0, 0)),
            pl.BlockSpec((512, D), lambda i: (0, 0)),
        ],
        out_specs=pl.BlockSpec((_SB, B, D), lambda i: (i, 0, 0)),
        out_shape=jax.ShapeDtypeStruct((S, B, D), x.dtype),
    )(xt, pos_table)
    return jnp.transpose(out_t, (1, 0, 2))


# speedup vs baseline: 1.1096x; 1.0018x over previous
import jax
import jax.numpy as jnp
from jax.experimental import pallas as pl

_SB = 24


def _posadd_kernel(x_ref, pos_ref, o_ref):
    i = pl.program_id(0)
    pos = pos_ref[pl.ds(i * _SB, _SB), :]
    o_ref[...] = x_ref[...] + pos[:, None, :]


def kernel(x, pos_table):
    B, S, D = x.shape
    xt = jnp.transpose(x, (1, 0, 2))
    out_t = pl.pallas_call(
        _posadd_kernel,
        grid=(pl.cdiv(S, _SB),),
        in_specs=[
            pl.BlockSpec((_SB, B, D), lambda i: (i, 0, 0)),
            pl.BlockSpec((512, D), lambda i: (0, 0)),
        ],
        out_specs=pl.BlockSpec((_SB, B, D), lambda i: (i, 0, 0)),
        out_shape=jax.ShapeDtypeStruct((S, B, D), x.dtype),
    )(xt, pos_table)
    return jnp.transpose(out_t, (1, 0, 2))
